# Initial kernel scaffold; baseline (speedup 1.0000x reference)
import jax
import jax.numpy as jnp
from jax.experimental import pallas as pl


def kernel(x, edge_index, W_self1, W_neigh1, b1, W_self2, W_neigh2, b2):
    raise NotImplementedError("write your pallas kernel here")


# speedup vs baseline: 5.7297x; 5.7297x over previous
import functools

import jax
import jax.numpy as jnp
from jax import lax
from jax.experimental import pallas as pl
from jax.experimental.pallas import tpu as pltpu
from jax.experimental.pallas import tpu_sc as plsc

F = 128
CHUNK = 128
ZR = 8


def _sc_segment_sum(table, src_idx, dst_idx, width, chunks_per_tile):
    nrow = table.shape[0]
    n = nrow - 1
    info = plsc.get_sparse_core_info()
    ncores, nsub = info.num_cores, info.num_subcores
    rows_per_tile = ((n + nsub) + (ZR * nsub - 1)) // (ZR * nsub) * ZR
    n_pad = rows_per_tile * nsub
    zcopies = rows_per_tile // ZR

    mesh = plsc.VectorSubcoreMesh(core_axis_name="c", subcore_axis_name="s")

    @functools.partial(
        pl.kernel,
        mesh=mesh,
        out_type=jax.ShapeDtypeStruct((ncores, n_pad, width), jnp.float32),
        scratch_types=[
            pltpu.VMEM((CHUNK,), jnp.int32),
            pltpu.VMEM((CHUNK,), jnp.int32),
            pltpu.VMEM((CHUNK, width), jnp.float32),
            pltpu.VMEM((ZR, width), jnp.float32),
            pltpu.VMEM_SHARED((n_pad, width), jnp.float32),
            pltpu.SemaphoreType.DMA,
        ],
        compiler_params=pltpu.CompilerParams(use_tc_tiling_on_sc=False),
    )
    def agg(table_hbm, src_hbm, dst_hbm, out_hbm, sidx, didx, rows, zbuf, acc, sem):
        c = lax.axis_index("c")
        s = lax.axis_index("s")
        tile = c * nsub + s

        for i in range(ZR):
            for j in range(width // 16):
                zbuf[i, pl.ds(j * 16, 16)] = jnp.zeros((16,), jnp.float32)

        row0 = s * rows_per_tile

        def zcopy(k, _):
            pltpu.sync_copy(zbuf, acc.at[pl.ds(row0 + k * ZR, ZR)])
            return 0
        lax.fori_loop(0, zcopies, zcopy, 0)
        plsc.subcore_barrier()

        ebase = tile * (chunks_per_tile * CHUNK)

        def body(i, _):
            b = ebase + i * CHUNK
            pltpu.sync_copy(src_hbm.at[pl.ds(b, CHUNK)], sidx)
            pltpu.sync_copy(dst_hbm.at[pl.ds(b, CHUNK)], didx)
            pltpu.async_copy(table_hbm.at[sidx], rows, sem).wait()
            pltpu.sync_copy(rows, acc.at[didx], add=True)
            return 0
        lax.fori_loop(0, chunks_per_tile, body, 0)
        plsc.subcore_barrier()

        pltpu.sync_copy(acc.at[pl.ds(row0, rows_per_tile)],
                        out_hbm.at[c, pl.ds(row0, rows_per_tile)])

    return agg(table, src_idx, dst_idx)


def _tc_layer1(x, p0, p1, w_self, w_neigh, b):
    n = x.shape[0]
    w1 = p0.shape[1]
    rblk = 1000
    grid = (n // rblk,)

    def body(x_ref, a0_ref, a1_ref, ws_ref, wn_ref, b_ref, o_ref, invd_ref):
        xv = x_ref[...]
        a0 = a0_ref[...]
        a1 = a1_ref[...]
        inv = 1.0 / (a0[:, F:F + 1] + a1[:, F:F + 1] + 1.0)
        mean = (a0[:, :F] + a1[:, :F] + xv) * inv
        h = jnp.dot(xv, ws_ref[...], preferred_element_type=jnp.float32)
        h = h + jnp.dot(mean, wn_ref[...], preferred_element_type=jnp.float32)
        h = h + b_ref[...]
        o_ref[...] = jnp.maximum(h, 0.0)
        invd_ref[...] = inv

    return pl.pallas_call(
        body,
        grid=grid,
        in_specs=[
            pl.BlockSpec((rblk, F), lambda i: (i, 0)),
            pl.BlockSpec((rblk, w1), lambda i: (i, 0)),
            pl.BlockSpec((rblk, w1), lambda i: (i, 0)),
            pl.BlockSpec((F, F), lambda i: (0, 0)),
            pl.BlockSpec((F, F), lambda i: (0, 0)),
            pl.BlockSpec((1, F), lambda i: (0, 0)),
        ],
        out_specs=[
            pl.BlockSpec((rblk, F), lambda i: (i, 0)),
            pl.BlockSpec((rblk, 1), lambda i: (i, 0)),
        ],
        out_shape=[
            jax.ShapeDtypeStruct((n, F), jnp.float32),
            jax.ShapeDtypeStruct((n, 1), jnp.float32),
        ],
    )(x, p0, p1, w_self, w_neigh, b.reshape(1, F))


def _tc_layer2(h, q0, q1, invd, w_self, w_neigh, b):
    n = h.shape[0]
    rblk = 1000
    grid = (n // rblk,)

    def body(h_ref, a0_ref, a1_ref, invd_ref, ws_ref, wn_ref, b_ref, o_ref):
        hv = h_ref[...]
        mean = (a0_ref[...] + a1_ref[...] + hv) * invd_ref[...]
        o = jnp.dot(hv, ws_ref[...], preferred_element_type=jnp.float32)
        o = o + jnp.dot(mean, wn_ref[...], preferred_element_type=jnp.float32)
        o_ref[...] = o + b_ref[...]

    return pl.pallas_call(
        body,
        grid=grid,
        in_specs=[
            pl.BlockSpec((rblk, F), lambda i: (i, 0)),
            pl.BlockSpec((rblk, F), lambda i: (i, 0)),
            pl.BlockSpec((rblk, F), lambda i: (i, 0)),
            pl.BlockSpec((rblk, 1), lambda i: (i, 0)),
            pl.BlockSpec((F, F), lambda i: (0, 0)),
            pl.BlockSpec((F, F), lambda i: (0, 0)),
            pl.BlockSpec((1, F), lambda i: (0, 0)),
        ],
        out_specs=pl.BlockSpec((rblk, F), lambda i: (i, 0)),
        out_shape=jax.ShapeDtypeStruct((n, F), jnp.float32),
    )(h, q0, q1, invd, w_self, w_neigh, b.reshape(1, F))


def kernel(x, edge_index, W_self1, W_neigh1, b1, W_self2, W_neigh2, b2):
    n = x.shape[0]
    src = edge_index[0].astype(jnp.int32)
    dst = edge_index[1].astype(jnp.int32)
    e = src.shape[0]

    ntiles = 32
    chunks_per_tile = pl.cdiv(e, ntiles * CHUNK)
    e_pad = ntiles * chunks_per_tile * CHUNK
    pad = e_pad - e
    src_p = jnp.concatenate([src, jnp.full((pad,), n, jnp.int32)])
    dst_p = jnp.concatenate([dst, jnp.full((pad,), n, jnp.int32)])

    w1 = F + 16
    xt = jnp.concatenate(
        [x, jnp.ones((n, 1), x.dtype), jnp.zeros((n, w1 - F - 1), x.dtype)], axis=1)
    xt = jnp.concatenate([xt, jnp.zeros((1, w1), x.dtype)], axis=0)

    p = _sc_segment_sum(xt, src_p, dst_p, w1, chunks_per_tile)
    h, invd = _tc_layer1(x, p[0, :n], p[1, :n], W_self1, W_neigh1, b1)

    ht = jnp.concatenate([h, jnp.zeros((1, F), h.dtype)], axis=0)
    q = _sc_segment_sum(ht, src_p, dst_p, F, chunks_per_tile)
    return _tc_layer2(h, q[0, :n], q[1, :n], invd, W_self2, W_neigh2, b2)
